# baseline (speedup 1.0000x reference)
import functools

import jax
import jax.numpy as jnp
from jax import lax
from jax.experimental import pallas as pl
from jax.experimental.pallas import tpu as pltpu
from jax.experimental.pallas import tpu_sc as plsc

_B, _LX, _LXR, _D = 16, 2048, 2048, 256
_SEQ_OUT = _LX + _LXR + 3
_TOUT = _B * _SEQ_OUT
_NV = _D // 16


def _m8(v):
    return pl.multiple_of(v, 8)


def _row_copy(dst, dt, src, st):
    for c in range(_NV):
        dst[dt, pl.ds(c * 16, 16)] = src[st, pl.ds(c * 16, 16)]


def _assemble_body(x_hbm, xr_hbm, cls_hbm, ring_hbm, end_hbm, out_hbm,
                   bufa0, bufa1, bufb, cbuf, rbuf, ebuf, gsem, ssem):
    c = lax.axis_index("c")
    s = lax.axis_index("s")
    wid = s * 2 + c
    i = wid // 2
    h = wid % 2

    k = (3 * i) % 8
    base = _SEQ_OUT * i - k
    p_x = 7 - k
    p_r = (6 - k) % 8
    xrow0 = i * _LX
    rrow0 = i * _LXR

    pltpu.sync_copy(cls_hbm, cbuf)
    pltpu.sync_copy(ring_hbm, rbuf)
    pltpu.sync_copy(end_hbm, ebuf)

    def start_gathers(j, buf):
        cps = []
        def g(ref, so, rows, bo):
            cp = pltpu.make_async_copy(ref.at[pl.ds(_m8(so), rows)],
                                       buf.at[pl.ds(bo, rows)], gsem)
            cp.start()
            cps.append(cp)
        if j == 0:
            g(xr_hbm, jnp.maximum(i - 1, 0) * _LXR + _LXR - 8, 8, 0)
            g(x_hbm, xrow0, 128, 8)
        elif 1 <= j <= 15:
            g(x_hbm, xrow0 + 128 * j - 8, 136, 0)
        elif j == 16:
            g(x_hbm, xrow0 + _LX - 8, 8, 0)
            g(xr_hbm, rrow0, 128, 8)
        elif 17 <= j <= 31:
            g(xr_hbm, rrow0 + 128 * j - 2050 - k - p_r, 136, 0)
        else:
            g(xr_hbm, rrow0 + _LXR - 16, 16, 0)
        return cps

    def assemble(j, buf):
        if j == 0:
            def body(t4, carry):
                for u in range(4):
                    t = t4 * 4 + u
                    si = t - k + 7 + jnp.where(t <= k - 2, 2, 0)
                    _row_copy(bufb, t, buf, si)
                return carry
            lax.fori_loop(0, 32, body, 0)
            @pl.when(k >= 1)
            def _():
                _row_copy(bufb, k - 1, ebuf, 0)
            _row_copy(bufb, k, cbuf, 0)
        elif 1 <= j <= 15:
            def body(t4, carry):
                for u in range(4):
                    t = t4 * 4 + u
                    _row_copy(bufb, t, buf, t + p_x)
                return carry
            lax.fori_loop(0, 32, body, 0)
        elif j == 16:
            def body(t4, carry):
                for u in range(4):
                    t = t4 * 4 + u
                    si = t - k + 6 + jnp.where(t <= k, 1, 0)
                    _row_copy(bufb, t, buf, si)
                return carry
            lax.fori_loop(0, 32, body, 0)
            _row_copy(bufb, k + 1, rbuf, 0)
        elif 17 <= j <= 31:
            def body(t4, carry):
                for u in range(4):
                    t = t4 * 4 + u
                    _row_copy(bufb, t, buf, t + p_r)
                return carry
            lax.fori_loop(0, 32, body, 0)
        else:
            def body(t, carry):
                _row_copy(bufb, t, buf, 14 - k + t)
                return carry
            lax.fori_loop(0, 8, body, 0)
            _row_copy(bufb, k + 2, ebuf, 0)

    def run_pipeline(J):
        bufas = [bufa0, bufa1]
        n = len(J)
        gcur = start_gathers(J[0], bufas[0])
        pend = None
        for idx in range(n):
            j = J[idx]
            for cp in gcur:
                cp.wait()
            if idx + 1 < n:
                gnext = start_gathers(J[idx + 1], bufas[(idx + 1) % 2])
            if pend is not None:
                pend.wait()
            assemble(j, bufas[idx % 2])
            if j == 32:
                @pl.when(k >= 5)
                def _():
                    cp = pltpu.make_async_copy(
                        bufb.at[pl.ds(0, 8)],
                        out_hbm.at[pl.ds(_m8(base + 4096), 8)], ssem)
                    cp.start()
                    cp.wait()
                pend = None
            else:
                cp = pltpu.make_async_copy(
                    bufb.at[pl.ds(0, 128)],
                    out_hbm.at[pl.ds(_m8(base + 128 * j), 128)], ssem)
                cp.start()
                pend = cp
            if idx + 1 < n:
                gcur = gnext
        if pend is not None:
            pend.wait()

    @pl.when(h == 0)
    def _():
        run_pipeline(list(range(0, 17)))

    @pl.when(h == 1)
    def _():
        run_pipeline(list(range(17, 33)))


def kernel(X, Xr, CLS, RING, END, cu_seqlens_X, cu_seqlens_Xr):
    nb = cu_seqlens_X.shape[0] - 1
    out_cu = (cu_seqlens_X + cu_seqlens_Xr
              + 3 * jnp.arange(nb + 1, dtype=jnp.int32)).astype(jnp.int32)

    mesh = plsc.VectorSubcoreMesh(core_axis_name="c", subcore_axis_name="s")
    run = functools.partial(
        pl.kernel,
        mesh=mesh,
        out_type=jax.ShapeDtypeStruct((_TOUT, _D), jnp.float32),
        scratch_types=[
            pltpu.VMEM((136, _D), jnp.float32),
            pltpu.VMEM((136, _D), jnp.float32),
            pltpu.VMEM((128, _D), jnp.float32),
            pltpu.VMEM((1, _D), jnp.float32),
            pltpu.VMEM((1, _D), jnp.float32),
            pltpu.VMEM((1, _D), jnp.float32),
            pltpu.SemaphoreType.DMA,
            pltpu.SemaphoreType.DMA,
        ],
    )(_assemble_body)
    out = run(X, Xr, CLS, RING, END)
    return out, out_cu


# speedup vs baseline: 2.4325x; 2.4325x over previous
import functools

import jax
import jax.numpy as jnp
from jax import lax
from jax.experimental import pallas as pl
from jax.experimental.pallas import tpu as pltpu
from jax.experimental.pallas import tpu_sc as plsc

_B, _LX, _LXR, _D = 16, 2048, 2048, 256
_SEQ_OUT = _LX + _LXR + 3
_TOUT = _B * _SEQ_OUT
_NV = _D // 16


def _m8(v):
    return pl.multiple_of(v, 8)


def _row_copy(dst, dt, src, st):
    for c in range(_NV):
        dst[dt, pl.ds(c * 16, 16)] = src[st, pl.ds(c * 16, 16)]


def _assemble_body(x_hbm, xr_hbm, cls_hbm, ring_hbm, end_hbm, out_hbm,
                   bufa0, bufa1, bufb, cbuf, rbuf, ebuf, gsem, ssem):
    c = lax.axis_index("c")
    s = lax.axis_index("s")
    wid = s * 2 + c
    i = wid // 2
    h = wid % 2

    k = (3 * i) % 8
    base = _SEQ_OUT * i - k
    p_x = 7 - k
    p_r = (6 - k) % 8
    xrow0 = i * _LX
    rrow0 = i * _LXR

    pltpu.sync_copy(cls_hbm, cbuf)
    pltpu.sync_copy(ring_hbm, rbuf)
    pltpu.sync_copy(end_hbm, ebuf)

    def start_gathers(j, buf):
        cps = []
        def g(ref, so, rows, bo):
            cp = pltpu.make_async_copy(ref.at[pl.ds(_m8(so), rows)],
                                       buf.at[pl.ds(bo, rows)], gsem)
            cp.start()
            cps.append(cp)
        if j == 0:
            g(xr_hbm, jnp.maximum(i - 1, 0) * _LXR + _LXR - 8, 8, 0)
            g(x_hbm, xrow0, 128, 8)
        elif 1 <= j <= 15:
            g(x_hbm, xrow0 + 128 * j - 8, 136, 0)
        elif j == 16:
            g(x_hbm, xrow0 + _LX - 8, 8, 0)
            g(xr_hbm, rrow0, 128, 8)
        elif 17 <= j <= 31:
            g(xr_hbm, rrow0 + 128 * j - 2050 - k - p_r, 136, 0)
        else:
            g(xr_hbm, rrow0 + _LXR - 16, 16, 0)
        return cps

    def assemble(j, buf):
        if j == 0:
            @functools.partial(plsc.parallel_loop, 0, 128, unroll=4)
            def _(t):
                si = t - k + 7 + jnp.where(t <= k - 2, 2, 0)
                _row_copy(bufb, t, buf, si)
            @pl.when(k >= 1)
            def _():
                _row_copy(bufb, k - 1, ebuf, 0)
            _row_copy(bufb, k, cbuf, 0)
        elif 1 <= j <= 15:
            @functools.partial(plsc.parallel_loop, 0, 128, unroll=4)
            def _(t):
                _row_copy(bufb, t, buf, t + p_x)
        elif j == 16:
            @functools.partial(plsc.parallel_loop, 0, 128, unroll=4)
            def _(t):
                si = t - k + 6 + jnp.where(t <= k, 1, 0)
                _row_copy(bufb, t, buf, si)
            _row_copy(bufb, k + 1, rbuf, 0)
        elif 17 <= j <= 31:
            @functools.partial(plsc.parallel_loop, 0, 128, unroll=4)
            def _(t):
                _row_copy(bufb, t, buf, t + p_r)
        else:
            @functools.partial(plsc.parallel_loop, 0, 8, unroll=1)
            def _(t):
                _row_copy(bufb, t, buf, 14 - k + t)
            _row_copy(bufb, k + 2, ebuf, 0)

    def run_pipeline(J):
        bufas = [bufa0, bufa1]
        n = len(J)
        gcur = start_gathers(J[0], bufas[0])
        pend = None
        for idx in range(n):
            j = J[idx]
            for cp in gcur:
                cp.wait()
            if idx + 1 < n:
                gnext = start_gathers(J[idx + 1], bufas[(idx + 1) % 2])
            if pend is not None:
                pend.wait()
            assemble(j, bufas[idx % 2])
            if j == 32:
                @pl.when(k >= 5)
                def _():
                    cp = pltpu.make_async_copy(
                        bufb.at[pl.ds(0, 8)],
                        out_hbm.at[pl.ds(_m8(base + 4096), 8)], ssem)
                    cp.start()
                    cp.wait()
                pend = None
            else:
                cp = pltpu.make_async_copy(
                    bufb.at[pl.ds(0, 128)],
                    out_hbm.at[pl.ds(_m8(base + 128 * j), 128)], ssem)
                cp.start()
                pend = cp
            if idx + 1 < n:
                gcur = gnext
        if pend is not None:
            pend.wait()

    @pl.when(h == 0)
    def _():
        run_pipeline(list(range(0, 17)))

    @pl.when(h == 1)
    def _():
        run_pipeline(list(range(17, 33)))


def kernel(X, Xr, CLS, RING, END, cu_seqlens_X, cu_seqlens_Xr):
    nb = cu_seqlens_X.shape[0] - 1
    out_cu = (cu_seqlens_X + cu_seqlens_Xr
              + 3 * jnp.arange(nb + 1, dtype=jnp.int32)).astype(jnp.int32)

    mesh = plsc.VectorSubcoreMesh(core_axis_name="c", subcore_axis_name="s")
    run = functools.partial(
        pl.kernel,
        mesh=mesh,
        out_type=jax.ShapeDtypeStruct((_TOUT, _D), jnp.float32),
        scratch_types=[
            pltpu.VMEM((136, _D), jnp.float32),
            pltpu.VMEM((136, _D), jnp.float32),
            pltpu.VMEM((128, _D), jnp.float32),
            pltpu.VMEM((1, _D), jnp.float32),
            pltpu.VMEM((1, _D), jnp.float32),
            pltpu.VMEM((1, _D), jnp.float32),
            pltpu.SemaphoreType.DMA,
            pltpu.SemaphoreType.DMA,
        ],
    )(_assemble_body)
    out = run(X, Xr, CLS, RING, END)
    return out, out_cu
